# baseline (speedup 1.0000x reference)
import functools

import jax
import jax.numpy as jnp
from jax import lax
from jax.experimental import pallas as pl
from jax.experimental.pallas import tpu as pltpu
from jax.experimental.pallas import tpu_sc as plsc

WIDTH = 64
B_TOTAL = 16384 * 20
NUM_WORKERS = 32
B_PER_W = B_TOTAL // NUM_WORKERS
CHUNK = 128
N_CHUNKS = B_PER_W // CHUNK

_mesh = plsc.VectorSubcoreMesh(core_axis_name="c", subcore_axis_name="s")


@functools.partial(
    pl.kernel,
    out_type=jax.ShapeDtypeStruct((B_TOTAL, WIDTH), jnp.float32),
    mesh=_mesh,
    scratch_types=[
        pltpu.VMEM((N_CHUNKS, CHUNK), jnp.int32),
        pltpu.VMEM((CHUNK, WIDTH), jnp.float32),
        pltpu.SemaphoreType.DMA,
    ],
    compiler_params=pltpu.CompilerParams(use_tc_tiling_on_sc=False),
)
def _emb_gather(idx_hbm, table_hbm, out_hbm, idx_v, rows_v, sem):
    num_cores = 2
    wid = lax.axis_index("s") * num_cores + lax.axis_index("c")
    base = wid * B_PER_W
    pltpu.sync_copy(idx_hbm.at[wid], idx_v)

    def body(j, _):
        pltpu.async_copy(table_hbm.at[idx_v.at[j]], rows_v, sem).wait()
        pltpu.sync_copy(rows_v, out_hbm.at[pl.ds(base + j * CHUNK, CHUNK)])
        return 0

    lax.fori_loop(0, N_CHUNKS, body, 0)


def kernel(inputs, embeddings):
    idx = jnp.reshape(inputs.astype(jnp.int32), (NUM_WORKERS, N_CHUNKS, CHUNK))
    out = _emb_gather(idx, embeddings)
    return jnp.reshape(out, inputs.shape + (WIDTH,))


# speedup vs baseline: 1.0629x; 1.0629x over previous
import functools

import jax
import jax.numpy as jnp
from jax import lax
from jax.experimental import pallas as pl
from jax.experimental.pallas import tpu as pltpu
from jax.experimental.pallas import tpu_sc as plsc

WIDTH = 64
B_TOTAL = 16384 * 20
NUM_WORKERS = 32
B_PER_W = B_TOTAL // NUM_WORKERS
CHUNK = 128
N_CHUNKS = B_PER_W // CHUNK
NBUF = 4

_mesh = plsc.VectorSubcoreMesh(core_axis_name="c", subcore_axis_name="s")


@functools.partial(
    pl.kernel,
    out_type=jax.ShapeDtypeStruct((B_TOTAL, WIDTH), jnp.float32),
    mesh=_mesh,
    scratch_types=[
        pltpu.VMEM((N_CHUNKS, CHUNK), jnp.int32),
        pltpu.VMEM((NBUF, CHUNK, WIDTH), jnp.float32),
        pltpu.SemaphoreType.DMA,
    ],
    compiler_params=pltpu.CompilerParams(use_tc_tiling_on_sc=False),
)
def _emb_gather(idx_hbm, table_hbm, out_hbm, idx_v, rows_v, sem):
    num_cores = 2
    wid = lax.axis_index("s") * num_cores + lax.axis_index("c")
    base = wid * B_PER_W
    pltpu.sync_copy(idx_hbm.at[wid], idx_v)

    def fire(j, b):
        pltpu.async_copy(table_hbm.at[idx_v.at[j]], rows_v.at[b], sem)

    def drain_and_write(j, b):
        pltpu.make_async_copy(
            table_hbm.at[pl.ds(0, CHUNK)], rows_v.at[b], sem).wait()
        pltpu.sync_copy(rows_v.at[b], out_hbm.at[pl.ds(base + j * CHUNK, CHUNK)])

    for b in range(NBUF):
        fire(b, b)

    def body(gi, _):
        for b in range(NBUF):
            j = gi * NBUF + b
            drain_and_write(j, b)
            fire(j + NBUF, b)
        return 0

    lax.fori_loop(0, N_CHUNKS // NBUF - 1, body, 0)
    for b in range(NBUF):
        drain_and_write(N_CHUNKS - NBUF + b, b)


def kernel(inputs, embeddings):
    idx = jnp.reshape(inputs.astype(jnp.int32), (NUM_WORKERS, N_CHUNKS, CHUNK))
    out = _emb_gather(idx, embeddings)
    return jnp.reshape(out, inputs.shape + (WIDTH,))


# speedup vs baseline: 1.0645x; 1.0015x over previous
import functools

import jax
import jax.numpy as jnp
from jax import lax
from jax.experimental import pallas as pl
from jax.experimental.pallas import tpu as pltpu
from jax.experimental.pallas import tpu_sc as plsc

WIDTH = 64
B_TOTAL = 16384 * 20
NUM_WORKERS = 32
B_PER_W = B_TOTAL // NUM_WORKERS
CHUNK = 128
N_CHUNKS = B_PER_W // CHUNK
NBUF = 8
KD = 4

_mesh = plsc.VectorSubcoreMesh(core_axis_name="c", subcore_axis_name="s")


@functools.partial(
    pl.kernel,
    out_type=jax.ShapeDtypeStruct((B_TOTAL, WIDTH), jnp.float32),
    mesh=_mesh,
    scratch_types=[
        pltpu.VMEM((N_CHUNKS, CHUNK), jnp.int32),
        pltpu.VMEM((NBUF, CHUNK, WIDTH), jnp.float32),
        pltpu.SemaphoreType.DMA,
        pltpu.SemaphoreType.DMA,
    ],
    compiler_params=pltpu.CompilerParams(use_tc_tiling_on_sc=False),
)
def _emb_gather(idx_hbm, table_hbm, out_hbm, idx_v, rows_v, sem_g, sem_w):
    num_cores = 2
    wid = lax.axis_index("s") * num_cores + lax.axis_index("c")
    base = wid * B_PER_W
    pltpu.sync_copy(idx_hbm.at[wid], idx_v)

    def fire_gather(b, j):
        pltpu.async_copy(table_hbm.at[idx_v.at[j]], rows_v.at[b], sem_g)

    def wait_gather(b):
        pltpu.make_async_copy(
            table_hbm.at[pl.ds(0, CHUNK)], rows_v.at[b], sem_g).wait()

    def fire_write(b, j):
        pltpu.async_copy(
            rows_v.at[b], out_hbm.at[pl.ds(base + j * CHUNK, CHUNK)], sem_w)

    def wait_write(b):
        pltpu.make_async_copy(
            rows_v.at[b], out_hbm.at[pl.ds(base, CHUNK)], sem_w).wait()


    for b in range(NBUF):
        fire_gather(b, b)
        if b >= KD:
            bd = b - KD
            wait_gather(bd)
            fire_write(bd, bd)

    def body(gi, _):
        for b in range(NBUF):
            j = gi * NBUF + b
            wait_write(b)
            fire_gather(b, j)
            bd = (b - KD) % NBUF
            wait_gather(bd)
            fire_write(bd, j - KD)
        return 0

    lax.fori_loop(1, N_CHUNKS // NBUF, body, 0)

    for j in range(N_CHUNKS - KD, N_CHUNKS):
        b = j % NBUF
        wait_gather(b)
        fire_write(b, j)
    for b in range(NBUF):
        wait_write(b)


def kernel(inputs, embeddings):
    idx = jnp.reshape(inputs.astype(jnp.int32), (NUM_WORKERS, N_CHUNKS, CHUNK))
    out = _emb_gather(idx, embeddings)
    return jnp.reshape(out, inputs.shape + (WIDTH,))


# speedup vs baseline: 1.1163x; 1.0487x over previous
import functools

import jax
import jax.numpy as jnp
from jax import lax
from jax.experimental import pallas as pl
from jax.experimental.pallas import tpu as pltpu
from jax.experimental.pallas import tpu_sc as plsc

VOCAB = 1000000
WIDTH = 64
HALF = 32
B_TOTAL = 16384 * 20
NUM_WORKERS = 32
B_PER_W = B_TOTAL // NUM_WORKERS
CHUNK = 128
N_LCHUNKS = B_PER_W // CHUNK
N_CHUNKS = 2 * N_LCHUNKS
NBUF = 8
KD = 4

TBLK = 2048


def _transpose_body(src_ref, dst_ref):
    x = src_ref[...]
    xt = jnp.transpose(x, (1, 0))
    xp = jnp.reshape(xt, (TBLK // 2, 2, WIDTH))
    dst_ref[:, :WIDTH] = xp[:, 0, :]
    dst_ref[:, WIDTH:] = xp[:, 1, :]


_tc_transpose = pl.pallas_call(
    _transpose_body,
    grid=((VOCAB + TBLK - 1) // TBLK,),
    in_specs=[pl.BlockSpec((WIDTH, TBLK), lambda i: (0, i))],
    out_specs=pl.BlockSpec((TBLK // 2, 2 * WIDTH), lambda i: (i, 0)),
    out_shape=jax.ShapeDtypeStruct((VOCAB // 2, 2 * WIDTH), jnp.float32),
)

_mesh = plsc.VectorSubcoreMesh(core_axis_name="c", subcore_axis_name="s")


@functools.partial(
    pl.kernel,
    out_type=jax.ShapeDtypeStruct((B_TOTAL, 2, HALF), jnp.float32),
    mesh=_mesh,
    scratch_types=[
        pltpu.VMEM((B_PER_W,), jnp.int32),
        pltpu.VMEM((B_PER_W,), jnp.int32),
        pltpu.VMEM((B_PER_W,), jnp.int32),
        pltpu.VMEM((NBUF, CHUNK, HALF), jnp.float32),
        pltpu.SemaphoreType.DMA,
        pltpu.SemaphoreType.DMA,
    ],
    compiler_params=pltpu.CompilerParams(use_tc_tiling_on_sc=False),
)
def _emb_gather(idx_hbm, table_hbm, out_hbm, idx_v, pe_v, po_v, rows_v,
                sem_g, sem_w):
    num_cores = 2
    wid = lax.axis_index("s") * num_cores + lax.axis_index("c")
    base = wid * B_PER_W
    pltpu.sync_copy(idx_hbm.at[wid], idx_v)

    def expand(g, _):
        v = idx_v[pl.ds(g * 16, 16)]
        d = v * 2
        pe_v[pl.ds(g * 16, 16)] = d
        po_v[pl.ds(g * 16, 16)] = d + 1
        return 0

    lax.fori_loop(0, B_PER_W // 16, expand, 0)

    def fire_gather(b, c):
        src = pe_v if b % 2 == 0 else po_v
        pltpu.async_copy(
            table_hbm.at[src.at[pl.ds((c // 2) * CHUNK, CHUNK)]],
            rows_v.at[b], sem_g)

    def wait_gather(b):
        pltpu.make_async_copy(
            table_hbm.at[pl.ds(0, CHUNK)], rows_v.at[b], sem_g).wait()

    def fire_write(b, c):
        pltpu.async_copy(
            rows_v.at[b],
            out_hbm.at[pl.ds(base + (c // 2) * CHUNK, CHUNK), b % 2],
            sem_w)

    def wait_write(b):
        pltpu.make_async_copy(
            rows_v.at[b], out_hbm.at[pl.ds(base, CHUNK), 0], sem_w).wait()

    for b in range(NBUF):
        fire_gather(b, b)
        if b >= KD:
            bd = b - KD
            wait_gather(bd)
            fire_write(bd, bd)

    def body(gi, _):
        for b in range(NBUF):
            c = gi * NBUF + b
            wait_write(b)
            fire_gather(b, c)
            bd = (b - KD) % NBUF
            wait_gather(bd)
            fire_write(bd, c - KD)
        return 0

    lax.fori_loop(1, N_CHUNKS // NBUF, body, 0)

    for c in range(N_CHUNKS - KD, N_CHUNKS):
        b = c % NBUF
        wait_gather(b)
        fire_write(b, c)
    for b in range(NBUF):
        wait_write(b)


def kernel(inputs, embeddings):
    table_lin = _tc_transpose(jnp.transpose(embeddings))
    table32 = jnp.reshape(table_lin, (2 * VOCAB, HALF))
    idx2 = jnp.reshape(inputs.astype(jnp.int32), (NUM_WORKERS, B_PER_W))
    out = _emb_gather(idx2, table32)
    return jnp.reshape(out, inputs.shape + (WIDTH,))
